# baseline (speedup 1.0000x reference)
import functools

import jax
import jax.numpy as jnp
from jax import lax
from jax.experimental import pallas as pl
from jax.experimental.pallas import tpu as pltpu
from jax.experimental.pallas import tpu_sc as plsc

B = 4096
S = 200
D = 64
_INFO = plsc.get_sparse_core_info()
NC = _INFO.num_cores
NS = _INFO.num_subcores
NW = NC * NS
BPW = (B * S) // NW
CH = 128
G = BPW // CH
LANES = 16
DJ = D // LANES


def _body(ids_hbm, pos_hbm, table_hbm, out_hbm, idx_v, pos2_v, rows_v, gsem):
    c = lax.axis_index("c")
    s = lax.axis_index("s")
    wid = s * NC + c

    pltpu.sync_copy(ids_hbm.at[wid], idx_v)
    pltpu.sync_copy(pos_hbm, pos2_v.at[pl.ds(0, S)])
    pltpu.sync_copy(pos_hbm, pos2_v.at[pl.ds(S, S)])

    def chunk(g, carry):
        pltpu.async_copy(table_hbm.at[idx_v.at[g]], rows_v, gsem).wait()
        p0 = lax.rem(g * CH, S)

        def rowfn(r, c2):
            for j in range(DJ):
                sl = pl.ds(j * LANES, LANES)
                rows_v[r, sl] = rows_v[r, sl] + pos2_v[p0 + r, sl]
            return c2

        lax.fori_loop(0, CH, rowfn, 0)
        pltpu.sync_copy(rows_v, out_hbm.at[wid, g])
        return carry

    lax.fori_loop(0, G, chunk, 0)


@jax.jit
def _encoder(ids3, pos_embedding, embedding):
    mesh = plsc.VectorSubcoreMesh(core_axis_name="c", subcore_axis_name="s")
    fn = functools.partial(
        pl.kernel,
        mesh=mesh,
        out_type=jax.ShapeDtypeStruct((NW, G, CH, D), jnp.float32),
        scratch_types=[
            pltpu.VMEM((G, CH), jnp.int32),
            pltpu.VMEM((2 * S, D), jnp.float32),
            pltpu.VMEM((CH, D), jnp.float32),
            pltpu.SemaphoreType.DMA,
        ],
        compiler_params=pltpu.CompilerParams(use_tc_tiling_on_sc=False),
    )(_body)
    return fn(ids3, pos_embedding, embedding)


def kernel(input_ids, embedding, pos_embedding):
    ids3 = input_ids.astype(jnp.int32).reshape(NW, G, CH)
    out = _encoder(ids3, pos_embedding, embedding)
    return out.reshape(B, S, D)


# speedup vs baseline: 1.1547x; 1.1547x over previous
import functools

import jax
import jax.numpy as jnp
from jax import lax
from jax.experimental import pallas as pl
from jax.experimental.pallas import tpu as pltpu
from jax.experimental.pallas import tpu_sc as plsc

B = 4096
S = 200
D = 64
_INFO = plsc.get_sparse_core_info()
NC = _INFO.num_cores
NS = _INFO.num_subcores
NW = NC * NS
BPW = (B * S) // NW
CH = 128
G = BPW // CH
LANES = 16
DJ = D // LANES
NB = 6
LK = 4
RU = 4


def _body(ids_hbm, pos_hbm, table_hbm, out_hbm, idx_v, pos2_v, rows_v,
          gsem, osem):
    c = lax.axis_index("c")
    s = lax.axis_index("s")
    wid = s * NC + c

    pltpu.sync_copy(ids_hbm.at[wid], idx_v)
    pltpu.sync_copy(pos_hbm, pos2_v.at[pl.ds(0, S)])
    pltpu.sync_copy(pos_hbm, pos2_v.at[pl.ds(S, S)])

    def gather_start(h):
        b = lax.rem(h, NB)
        pltpu.async_copy(table_hbm.at[idx_v.at[h]], rows_v.at[b], gsem.at[b])

    def gather_wait(b):
        pltpu.make_async_copy(out_hbm.at[wid, 0], rows_v.at[b],
                              gsem.at[b]).wait()

    def add_pos(g, b):
        p0 = lax.rem(g * CH, S)

        def rowfn(r0, c2):
            for u in range(RU):
                r = r0 * RU + u
                for j in range(DJ):
                    sl = pl.ds(j * LANES, LANES)
                    rows_v[b, r, sl] = rows_v[b, r, sl] + pos2_v[p0 + r, sl]
            return c2

        lax.fori_loop(0, CH // RU, rowfn, 0)

    def out_start(g, b):
        pltpu.async_copy(rows_v.at[b], out_hbm.at[wid, g], osem.at[b])

    def out_wait(h):
        b = lax.rem(h, NB)
        pltpu.make_async_copy(rows_v.at[b], out_hbm.at[wid, 0],
                              osem.at[b]).wait()

    for h in range(LK):
        gather_start(h)
    for g in range(LK):
        b = g % NB
        gather_wait(b)
        add_pos(g, b)
        out_start(g, b)
        if g + LK >= NB:
            out_wait(g + LK - NB)
        gather_start(g + LK)

    def step(g, carry):
        b = lax.rem(g, NB)
        gather_wait(b)
        add_pos(g, b)
        out_start(g, b)
        out_wait(g - (NB - LK))
        gather_start(g + LK)
        return carry

    lax.fori_loop(LK, G - LK, step, 0)

    for g in range(G - LK, G):
        b = g % NB
        gather_wait(b)
        add_pos(g, b)
        out_start(g, b)
        out_wait(g - (NB - LK))
    for h in range(G - (NB - LK), G):
        out_wait(h)


@jax.jit
def _encoder(ids3, pos_embedding, embedding):
    mesh = plsc.VectorSubcoreMesh(core_axis_name="c", subcore_axis_name="s")
    fn = functools.partial(
        pl.kernel,
        mesh=mesh,
        out_type=jax.ShapeDtypeStruct((NW, G, CH, D), jnp.float32),
        scratch_types=[
            pltpu.VMEM((G, CH), jnp.int32),
            pltpu.VMEM((2 * S, D), jnp.float32),
            pltpu.VMEM((NB, CH, D), jnp.float32),
            pltpu.SemaphoreType.DMA((NB,)),
            pltpu.SemaphoreType.DMA((NB,)),
        ],
        compiler_params=pltpu.CompilerParams(use_tc_tiling_on_sc=False),
    )(_body)
    return fn(ids3, pos_embedding, embedding)


def kernel(input_ids, embedding, pos_embedding):
    ids3 = input_ids.astype(jnp.int32).reshape(NW, G, CH)
    out = _encoder(ids3, pos_embedding, embedding)
    return out.reshape(B, S, D)


# speedup vs baseline: 1.1942x; 1.0343x over previous
import functools

import jax
import jax.numpy as jnp
from jax import lax
from jax.experimental import pallas as pl
from jax.experimental.pallas import tpu as pltpu
from jax.experimental.pallas import tpu_sc as plsc

B = 4096
S = 200
D = 64
_INFO = plsc.get_sparse_core_info()
NC = _INFO.num_cores
NS = _INFO.num_subcores
NW = NC * NS
SPW = B // NW
CH0 = 128
CH1 = S - CH0
LANES = 16
DJ = D // LANES
NB = 6
LK = 4
RU = 4


def _body(ids_hbm, pos_hbm, table_hbm, out_hbm, idx_v, pos_v, rows_v,
          gsem, osem):
    c = lax.axis_index("c")
    s = lax.axis_index("s")
    wid = s * NC + c
    seq0 = wid * SPW

    pltpu.sync_copy(ids_hbm.at[pl.ds(seq0, SPW)], idx_v)
    pltpu.sync_copy(pos_hbm, pos_v)

    def gather_start(h):
        b = lax.rem(h, NB)
        pltpu.async_copy(table_hbm.at[idx_v.at[h, pl.ds(0, CH0)]],
                         rows_v.at[b, pl.ds(0, CH0)], gsem.at[b])
        pltpu.async_copy(table_hbm.at[idx_v.at[h, pl.ds(CH0, CH1)]],
                         rows_v.at[b, pl.ds(CH0, CH1)], gsem.at[b])

    def gather_wait(b):
        pltpu.make_async_copy(out_hbm.at[seq0], rows_v.at[b],
                              gsem.at[b]).wait()

    def add_pos(b):
        def rowfn(r0, c2):
            for u in range(RU):
                r = r0 * RU + u
                for j in range(DJ):
                    sl = pl.ds(j * LANES, LANES)
                    rows_v[b, r, sl] = rows_v[b, r, sl] + pos_v[r, sl]
            return c2

        lax.fori_loop(0, S // RU, rowfn, 0)

    def out_start(h, b):
        pltpu.async_copy(rows_v.at[b], out_hbm.at[seq0 + h], osem.at[b])

    def out_wait(h):
        b = lax.rem(h, NB)
        pltpu.make_async_copy(rows_v.at[b], out_hbm.at[seq0],
                              osem.at[b]).wait()

    for h in range(LK):
        gather_start(h)
    for g in range(LK):
        b = g % NB
        gather_wait(b)
        add_pos(b)
        out_start(g, b)
        if g + LK >= NB:
            out_wait(g + LK - NB)
        gather_start(g + LK)

    def step(g, carry):
        b = lax.rem(g, NB)
        gather_wait(b)
        add_pos(b)
        out_start(g, b)
        out_wait(g - (NB - LK))
        gather_start(g + LK)
        return carry

    lax.fori_loop(LK, SPW - LK, step, 0)

    for g in range(SPW - LK, SPW):
        b = g % NB
        gather_wait(b)
        add_pos(b)
        out_start(g, b)
        out_wait(g - (NB - LK))
    for h in range(SPW - (NB - LK), SPW):
        out_wait(h)


@jax.jit
def _encoder(ids, pos_embedding, embedding):
    mesh = plsc.VectorSubcoreMesh(core_axis_name="c", subcore_axis_name="s")
    fn = functools.partial(
        pl.kernel,
        mesh=mesh,
        out_type=jax.ShapeDtypeStruct((B, S, D), jnp.float32),
        scratch_types=[
            pltpu.VMEM((SPW, S), jnp.int32),
            pltpu.VMEM((S, D), jnp.float32),
            pltpu.VMEM((NB, S, D), jnp.float32),
            pltpu.SemaphoreType.DMA((NB,)),
            pltpu.SemaphoreType.DMA((NB,)),
        ],
        compiler_params=pltpu.CompilerParams(use_tc_tiling_on_sc=False),
    )(_body)
    return fn(ids, pos_embedding, embedding)


def kernel(input_ids, embedding, pos_embedding):
    return _encoder(input_ids.astype(jnp.int32), pos_embedding, embedding)
